# baseline (speedup 1.0000x reference)
import functools

import jax
import jax.numpy as jnp
from jax import lax
from jax.experimental import pallas as pl
from jax.experimental.pallas import tpu as pltpu
from jax.experimental.pallas import tpu_sc as plsc

_NUM_LAYERS = 24
_STM = 2048
_D = 1024
_NC = 2
_NS = 16
_NW = _NC * _NS
_RPW = _STM // _NW
_NCH = 8
_CR = _RPW // _NCH
_L = 16

_mesh = plsc.VectorSubcoreMesh(core_axis_name="c", subcore_axis_name="s")


@functools.partial(
    pl.kernel,
    mesh=_mesh,
    out_type=jax.ShapeDtypeStruct((1, _STM, _D), jnp.float32),
    scratch_types=[
        pltpu.VMEM((_L,), jnp.int32),
        pltpu.VMEM((_RPW, _D), jnp.float32),
        [pltpu.SemaphoreType.DMA] * _NCH,
        pltpu.SemaphoreType.DMA,
    ],
)
def _stm_lookup(mem_hbm, layer_hbm, out_hbm, lvm, rows_v, gsems, ssem):
    wid = lax.axis_index("s") * _NC + lax.axis_index("c")
    base = wid * _RPW
    pltpu.sync_copy(layer_hbm, lvm)
    lay = lax.squeeze(lax.slice(lvm[...], (0,), (1,)), (0,))
    gets = []
    for j in range(_NCH):
        c = pltpu.async_copy(
            mem_hbm.at[lay, pl.ds(base + j * _CR, _CR)],
            rows_v.at[pl.ds(j * _CR, _CR)],
            gsems[j],
        )
        gets.append(c)
    puts = []
    for j in range(_NCH):
        gets[j].wait()
        c = pltpu.async_copy(
            rows_v.at[pl.ds(j * _CR, _CR)],
            out_hbm.at[0, pl.ds(base + j * _CR, _CR)],
            ssem,
        )
        puts.append(c)
    for c in puts:
        c.wait()


def kernel(memory, layer):
    layer_vec = jnp.full((_L,), layer, dtype=jnp.int32)
    return _stm_lookup(memory, layer_vec)


# speedup vs baseline: 1.0311x; 1.0311x over previous
import functools

import jax
import jax.numpy as jnp
from jax import lax
from jax.experimental import pallas as pl
from jax.experimental.pallas import tpu as pltpu
from jax.experimental.pallas import tpu_sc as plsc

_NUM_LAYERS = 24
_STM = 2048
_D = 1024
_NC = 2
_NS = 16
_NW = _NC * _NS
_RPW = _STM // _NW
_NCH = 8
_CR = _RPW // _NCH
_L = 16

_mesh = plsc.VectorSubcoreMesh(core_axis_name="c", subcore_axis_name="s")


@functools.partial(
    pl.kernel,
    mesh=_mesh,
    out_type=jax.ShapeDtypeStruct((1, _STM, _D), jnp.float32),
    scratch_types=[
        pltpu.VMEM((_L,), jnp.int32),
        pltpu.VMEM((_RPW, _D), jnp.float32),
    ],
)
def _stm_lookup(mem_hbm, layer_hbm, out_hbm, lvm, rows_v):
    wid = lax.axis_index("s") * _NC + lax.axis_index("c")
    base = wid * _RPW
    pltpu.sync_copy(layer_hbm, lvm)
    lay = lax.squeeze(lax.slice(lvm[...], (0,), (1,)), (0,))
    pltpu.sync_copy(mem_hbm.at[lay, pl.ds(base, _RPW)], rows_v)
    pltpu.sync_copy(rows_v, out_hbm.at[0, pl.ds(base, _RPW)])


def kernel(memory, layer):
    layer_vec = jnp.full((_L,), layer, dtype=jnp.int32)
    return _stm_lookup(memory, layer_vec)


# speedup vs baseline: 1.0315x; 1.0003x over previous
import functools

import jax
import jax.numpy as jnp
from jax import lax
from jax.experimental import pallas as pl
from jax.experimental.pallas import tpu as pltpu
from jax.experimental.pallas import tpu_sc as plsc

_NUM_LAYERS = 24
_STM = 2048
_D = 1024
_NC = 2
_RPS = _STM // _NC
_NCH = 16
_CR = _RPS // _NCH
_L = 16

_mesh = plsc.ScalarSubcoreMesh(axis_name="c")


@functools.partial(
    pl.kernel,
    mesh=_mesh,
    out_type=jax.ShapeDtypeStruct((1, _STM, _D), jnp.float32),
    scratch_types=[
        pltpu.SMEM((_L,), jnp.int32),
        pltpu.VMEM_SHARED((_RPS, _D), jnp.float32),
        [pltpu.SemaphoreType.DMA] * _NCH,
        pltpu.SemaphoreType.DMA,
    ],
)
def _stm_lookup(mem_hbm, layer_hbm, out_hbm, lsm, rows_sp, gsems, ssem):
    cid = lax.axis_index("c")
    base = cid * _RPS
    pltpu.sync_copy(layer_hbm, lsm)
    lay = lsm[0]
    gets = []
    for j in range(_NCH):
        c = pltpu.async_copy(
            mem_hbm.at[lay, pl.ds(base + j * _CR, _CR)],
            rows_sp.at[pl.ds(j * _CR, _CR)],
            gsems[j],
        )
        gets.append(c)
    puts = []
    for j in range(_NCH):
        gets[j].wait()
        c = pltpu.async_copy(
            rows_sp.at[pl.ds(j * _CR, _CR)],
            out_hbm.at[0, pl.ds(base + j * _CR, _CR)],
            ssem,
        )
        puts.append(c)
    for c in puts:
        c.wait()


def kernel(memory, layer):
    layer_vec = jnp.full((_L,), layer, dtype=jnp.int32)
    return _stm_lookup(memory, layer_vec)
